# baseline (speedup 1.0000x reference)
import functools

import jax
import jax.numpy as jnp
from jax.experimental import pallas as pl
from jax.experimental.pallas import tpu as pltpu

_SMOOTH = 0.1
_ROW_BLOCK = 64
_LANE = 128


def _loss_kernel(tgt_smem, pred_ref, *strips_out, num_classes, batch):
    strips = strips_out[:-1]
    out_ref = strips_out[-1]
    i = pl.program_id(0)
    rb = pred_ref.shape[0]

    x = pred_ref[...]
    m = jnp.max(x, axis=1, keepdims=True)
    s_sum = jnp.sum(x, axis=1, keepdims=True)
    e_sum = jnp.sum(jnp.exp(x - m), axis=1, keepdims=True)
    lse = jnp.log(e_sum)

    eps = _SMOOTH / (num_classes - 1)
    coef = 1.0 - _SMOOTH - eps
    vec_part = -(
        eps * (s_sum - num_classes * (m + lse)) + coef * (-m - lse)
    )

    lane = jax.lax.broadcasted_iota(jnp.int32, (1, _LANE), 1)
    pt_total = 0.0
    for j in range(rb):
        t = tgt_smem[i * rb + j]
        off = jax.lax.rem(t, _LANE)
        row = strips[j][j % 8, :].reshape(1, _LANE)
        pt_total += jnp.sum(jnp.where(lane == off, row, 0.0))

    block_sum = jnp.sum(vec_part) - coef * pt_total

    @pl.when(i == 0)
    def _():
        out_ref[...] = jnp.zeros((1, 1), jnp.float32)

    out_ref[...] += block_sum.reshape(1, 1) / batch


def _strip_spec(j, rb):
    def index_map(i, tref):
        r = i * rb + j
        return (r // 8, tref[r] // _LANE)

    return pl.BlockSpec((8, _LANE), index_map)


def kernel(pred, target):
    batch, num_classes = pred.shape
    tgt = target.astype(jnp.int32)
    rb = _ROW_BLOCK
    grid = batch // rb

    grid_spec = pltpu.PrefetchScalarGridSpec(
        num_scalar_prefetch=1,
        grid=(grid,),
        in_specs=[
            pl.BlockSpec((rb, num_classes), lambda i, tref: (i, 0)),
            *[_strip_spec(j, rb) for j in range(rb)],
        ],
        out_specs=pl.BlockSpec((1, 1), lambda i, tref: (0, 0)),
    )
    out = pl.pallas_call(
        functools.partial(_loss_kernel, num_classes=num_classes, batch=batch),
        grid_spec=grid_spec,
        out_shape=jax.ShapeDtypeStruct((1, 1), jnp.float32),
    )(tgt, pred, *([pred] * rb))
    return out[0, 0]
